# baseline (speedup 1.0000x reference)
import functools

import jax
import jax.numpy as jnp
from jax import lax
from jax.experimental import pallas as pl
from jax.experimental.pallas import tpu as pltpu
from jax.experimental.pallas import tpu_sc as plsc

EPS = 1e-05
S = 2048
H = 2048
I = 4096
E = 8
K = 2
NP = K * S

TBR = 256
GR = S // TBR

TB = 256
NBLK = 23
CAP = NBLK * TB
NBCAP = 8
ICF = 512
NIF = I // ICF
NSTEPS = 192

TSUB = 512
ICS = 512
NIS = I // ICS

NC = 2
NS = 16
NW = NC * NS
PPT = NP // NW
TPT = S // NW


def _router_body(x_ref, g_ref, b_ref, rw_ref, bias_ref,
                 eid_ref, pw_ref, z_ref):
    b = pl.program_id(0)
    x = x_ref[...]
    m = jnp.mean(x, axis=1, keepdims=True)
    v = jnp.mean((x - m) ** 2, axis=1, keepdims=True)
    xn = (x - m) / jnp.sqrt(v + 1e-05) * g_ref[...] + b_ref[...]
    logits = jax.lax.dot_general(xn, rw_ref[...],
                                 (((1,), (1,)), ((), ())),
                                 preferred_element_type=jnp.float32)
    logits = logits + bias_ref[...]
    lmax = jnp.max(logits, axis=1, keepdims=True)
    ex = jnp.exp(logits - lmax)
    sex = jnp.sum(ex, axis=1, keepdims=True)
    lse = lmax + jnp.log(sex)
    p = ex / sex

    iota = lax.broadcasted_iota(jnp.int32, (TBR, E), 1)
    m1 = jnp.max(p, axis=1, keepdims=True)
    i1 = jnp.min(jnp.where(p == m1, iota, E), axis=1, keepdims=True)
    p2 = jnp.where(iota == i1, -jnp.inf, p)
    m2 = jnp.max(p2, axis=1, keepdims=True)
    i2 = jnp.min(jnp.where(p2 == m2, iota, E), axis=1, keepdims=True)
    ssum = jnp.clip(m1 + m2, EPS, None)
    w1 = m1 / ssum
    w2 = m2 / ssum
    eid_ref[0, pl.ds(b * TBR, TBR)] = i1[:, 0]
    eid_ref[1, pl.ds(b * TBR, TBR)] = i2[:, 0]
    pw_ref[0, pl.ds(b * TBR, TBR)] = w1[:, 0]
    pw_ref[1, pl.ds(b * TBR, TBR)] = w2[:, 0]
    zrow = jnp.sum(lse * lse)
    lane = lax.broadcasted_iota(jnp.int32, (1, 128), 1)
    z_ref[pl.ds(b, 1), :] = jnp.where(lane == 0, zrow, 0.0)


def _router(x, ln_g, ln_b, router_w, expert_bias):
    return pl.pallas_call(
        _router_body,
        grid=(GR,),
        in_specs=[
            pl.BlockSpec((TBR, H), lambda b: (b, 0)),
            pl.BlockSpec((1, H), lambda b: (0, 0)),
            pl.BlockSpec((1, H), lambda b: (0, 0)),
            pl.BlockSpec((E, H), lambda b: (0, 0)),
            pl.BlockSpec((1, E), lambda b: (0, 0)),
        ],
        out_specs=[
            pl.BlockSpec((K, S), lambda b: (0, 0)),
            pl.BlockSpec((K, S), lambda b: (0, 0)),
            pl.BlockSpec((GR, 128), lambda b: (0, 0)),
        ],
        out_shape=[
            jax.ShapeDtypeStruct((K, S), jnp.int32),
            jax.ShapeDtypeStruct((K, S), jnp.float32),
            jax.ShapeDtypeStruct((GR, 128), jnp.float32),
        ],
    )(x, ln_g.reshape(1, H), ln_b.reshape(1, H), router_w,
      expert_bias.reshape(1, E))


def _dispatch_body(x_hbm, eid_hbm, xg_hbm, row_hbm, sched_hbm,
                   eid_v, pos_v, schedv,
                   tok0, tok1, tok2, tok3, dst0, dst1, dst2, dst3,
                   xbuf, sem):
    wid = lax.axis_index("s") * NC + lax.axis_index("c")
    iota16 = lax.iota(jnp.int32, 16)
    pltpu.sync_copy(eid_hbm, eid_v)

    def scan_body(k, cnt):
        ev = eid_v[pl.ds(k * 16, 16)]
        base = cnt.at[ev].get(mode="promise_in_bounds")
        rank = jnp.zeros((16,), jnp.int32)
        newcnt = cnt
        for e in range(E):
            m = ev == e
            inc = jnp.where(m, 1, 0).astype(jnp.int32)
            cs = jnp.cumsum(inc)
            rank = rank + jnp.where(m, cs, 0)
            tot = jnp.sum(inc)
            newcnt = newcnt + jnp.where(iota16 == e, tot, 0)
        pos_v[pl.ds(k * 16, 16)] = base + rank - 1
        return newcnt

    cnt = lax.fori_loop(0, NP // 16, scan_body,
                        jnp.zeros((16,), jnp.int32))
    aligned = ((cnt + (TB - 1)) // TB) * TB
    incl = jnp.cumsum(aligned)
    start = incl - aligned

    pbase = wid * PPT
    tok = (tok0, tok1, tok2, tok3)
    dst = (dst0, dst1, dst2, dst3)
    for c in range(4):
        for hh in range(2):
            off = pbase + c * 32 + hh * 16
            ev = eid_v[pl.ds(off, 16)]
            posv = pos_v[pl.ds(off, 16)]
            destv = start.at[ev].get(mode="promise_in_bounds") + posv
            dst[c][pl.ds(hh * 16, 16)] = destv
            pv = off + iota16
            tok[c][pl.ds(hh * 16, 16)] = jnp.bitwise_and(pv, S - 1)
        pltpu.async_copy(x_hbm.at[tok[c]], xbuf, sem).wait()
        pltpu.async_copy(xbuf, xg_hbm.at[dst[c]], sem).wait()
        pltpu.sync_copy(dst[c], row_hbm.at[pl.ds(pbase + c * 32, 32)])

    @pl.when(wid == 0)
    def _():
        nb_vec = aligned // TB
        bstart_vec = start // TB
        e_of_g = iota16 // 2
        sg_of_g = jnp.bitwise_and(iota16, 1)
        nb_g = nb_vec.at[e_of_g].get(mode="promise_in_bounds")
        bb_g = (bstart_vec.at[e_of_g].get(mode="promise_in_bounds")
                + sg_of_g * NBCAP)
        nbs_g = jnp.clip(nb_g - sg_of_g * NBCAP, 0, NBCAP)
        steps_g = NIF * nbs_g
        cum_g = jnp.cumsum(steps_g)
        segstart_g = cum_g - steps_g
        total = cum_g[15]
        laste = jnp.int32(0)
        lastb = jnp.int32(0)
        lastblk = jnp.int32(0)
        for g in range(16):
            ne = nbs_g[g] > 0
            laste = jnp.where(ne, jnp.int32(g // 2), laste)
            lastb = jnp.where(ne, nbs_g[g] - 1, lastb)
            lastblk = jnp.where(ne, bb_g[g] + nbs_g[g] - 1, lastblk)
        def colbody(ci, carry):
            sv = ci * 16 + iota16
            e_col = jnp.zeros((16,), jnp.int32)
            i_col = jnp.zeros((16,), jnp.int32)
            b_col = jnp.zeros((16,), jnp.int32)
            blk_col = jnp.zeros((16,), jnp.int32)
            for g in range(16):
                ss = segstart_g[g]
                nbs_s = jnp.maximum(nbs_g[g], 1)
                m = (sv >= ss) & (sv < ss + NIF * nbs_g[g])
                k = sv - ss
                i_val = k // nbs_s
                b_val = k - i_val * nbs_s
                e_col = jnp.where(m, jnp.int32(g // 2), e_col)
                i_col = jnp.where(m, i_val, i_col)
                b_col = jnp.where(m, b_val, b_col)
                blk_col = jnp.where(m, bb_g[g] + b_val, blk_col)
            valid = sv < total
            sl = pl.ds(ci * 16, 16)
            schedv[0, sl] = jnp.where(valid, e_col, laste)
            schedv[1, sl] = jnp.where(valid, i_col, NIF - 1)
            schedv[2, sl] = jnp.where(valid, b_col, lastb)
            schedv[3, sl] = jnp.where(valid, blk_col, lastblk)
            schedv[4, sl] = jnp.where(valid, 1, 0)
            return carry

        lax.fori_loop(0, NSTEPS // 16, colbody, 0)
        pltpu.sync_copy(schedv, sched_hbm)


def _dispatch(x, eid_flat):
    mesh = plsc.VectorSubcoreMesh(core_axis_name="c", subcore_axis_name="s")
    f = pl.kernel(
        _dispatch_body,
        out_type=[
            jax.ShapeDtypeStruct((CAP, H), jnp.float32),
            jax.ShapeDtypeStruct((NP,), jnp.int32),
            jax.ShapeDtypeStruct((5, NSTEPS), jnp.int32),
        ],
        mesh=mesh,
        scratch_types=[
            pltpu.VMEM((NP,), jnp.int32),
            pltpu.VMEM((NP,), jnp.int32),
            pltpu.VMEM((5, NSTEPS), jnp.int32),
            pltpu.VMEM((32,), jnp.int32),
            pltpu.VMEM((32,), jnp.int32),
            pltpu.VMEM((32,), jnp.int32),
            pltpu.VMEM((32,), jnp.int32),
            pltpu.VMEM((32,), jnp.int32),
            pltpu.VMEM((32,), jnp.int32),
            pltpu.VMEM((32,), jnp.int32),
            pltpu.VMEM((32,), jnp.int32),
            pltpu.VMEM((32, H), jnp.float32),
            pltpu.SemaphoreType.DMA,
        ],
        compiler_params=pltpu.CompilerParams(needs_layout_passes=False),
    )
    return f(x, eid_flat)


def _ffn_chunk(x, gw, uw, dw):
    xb = x.astype(jnp.bfloat16)
    g = jax.lax.dot_general(xb, gw.astype(jnp.bfloat16),
                            (((1,), (1,)), ((), ())),
                            preferred_element_type=jnp.float32)
    g = g * jax.nn.sigmoid(g)
    u = jax.lax.dot_general(xb, uw.astype(jnp.bfloat16),
                            (((1,), (1,)), ((), ())),
                            preferred_element_type=jnp.float32)
    h = g * u
    return jax.lax.dot_general(h.astype(jnp.bfloat16),
                               dw.astype(jnp.bfloat16),
                               (((1,), (1,)), ((), ())),
                               preferred_element_type=jnp.float32)


def _gffn_body(sched_ref, xg_ref, gw_ref, uw_ref, dw_ref, yg_ref, acc, sem):
    s = pl.program_id(0)
    i = sched_ref[1, s]
    ba = sched_ref[2, s]
    blk = sched_ref[3, s]
    valid = sched_ref[4, s]

    @pl.when(valid == 1)
    def _():
        y = _ffn_chunk(xg_ref[...], gw_ref[0], uw_ref[0], dw_ref[0])
        row = pl.multiple_of(ba * TB, TB)

        @pl.when(i == 0)
        def _():
            acc[pl.ds(row, TB), :] = y

        @pl.when(i > 0)
        def _():
            acc[pl.ds(row, TB), :] = acc[pl.ds(row, TB), :] + y

        @pl.when(i == NIF - 1)
        def _():
            orow = pl.multiple_of(blk * TB, TB)
            cp = pltpu.make_async_copy(
                acc.at[pl.ds(row, TB), :],
                yg_ref.at[pl.ds(orow, TB), :], sem)
            cp.start()
            cp.wait()


def _gffn(sched, xg, gate_w, up_w, down_w):
    grid_spec = pltpu.PrefetchScalarGridSpec(
        num_scalar_prefetch=1,
        grid=(NSTEPS,),
        in_specs=[
            pl.BlockSpec((TB, H), lambda s, sc: (sc[3, s], 0)),
            pl.BlockSpec((1, ICF, H), lambda s, sc: (sc[0, s], sc[1, s], 0)),
            pl.BlockSpec((1, ICF, H), lambda s, sc: (sc[0, s], sc[1, s], 0)),
            pl.BlockSpec((1, H, ICF), lambda s, sc: (sc[0, s], 0, sc[1, s])),
        ],
        out_specs=pl.BlockSpec(memory_space=pl.ANY),
        scratch_shapes=[
            pltpu.VMEM((NBCAP * TB, H), jnp.float32),
            pltpu.SemaphoreType.DMA,
        ],
    )
    return pl.pallas_call(
        _gffn_body,
        grid_spec=grid_spec,
        out_shape=jax.ShapeDtypeStruct((CAP, H), jnp.float32),
        compiler_params=pltpu.CompilerParams(
            vmem_limit_bytes=60 * 1024 * 1024),
    )(sched, xg, gate_w, up_w, down_w)


def _shared_body(x_ref, g_ref, b_ref, gw_ref, uw_ref, dw_ref, sg_ref,
                 out_ref):
    i = pl.program_id(0)
    for ts in range(S // TSUB):
        sl = pl.ds(ts * TSUB, TSUB)
        x = x_ref[sl, :]
        m = jnp.mean(x, axis=1, keepdims=True)
        v = jnp.mean((x - m) ** 2, axis=1, keepdims=True)
        sx = (x - m) / jnp.sqrt(v + 1e-05) * g_ref[...] + b_ref[...]
        y = _ffn_chunk(sx, gw_ref[...], uw_ref[...], dw_ref[...])

        @pl.when(i == 0)
        def _():
            out_ref[sl, :] = y

        @pl.when(i > 0)
        def _():
            out_ref[sl, :] = out_ref[sl, :] + y

        @pl.when(i == NIS - 1)
        def _():
            sig = jax.nn.sigmoid(sg_ref[0, 0])
            out_ref[sl, :] = out_ref[sl, :] * sig


def _shared(x, s_ln_g, s_ln_b, s_gate_w, s_up_w, s_down_w, shared_gate):
    return pl.pallas_call(
        _shared_body,
        grid=(NIS,),
        in_specs=[
            pl.BlockSpec((S, H), lambda i: (0, 0)),
            pl.BlockSpec((1, H), lambda i: (0, 0)),
            pl.BlockSpec((1, H), lambda i: (0, 0)),
            pl.BlockSpec((ICS, H), lambda i: (i, 0)),
            pl.BlockSpec((ICS, H), lambda i: (i, 0)),
            pl.BlockSpec((H, ICS), lambda i: (0, i)),
            pl.BlockSpec((1, 1), lambda i: (0, 0)),
        ],
        out_specs=pl.BlockSpec((S, H), lambda i: (0, 0)),
        out_shape=jax.ShapeDtypeStruct((S, H), jnp.float32),
        compiler_params=pltpu.CompilerParams(
            vmem_limit_bytes=62 * 1024 * 1024),
    )(x, s_ln_g.reshape(1, H), s_ln_b.reshape(1, H), s_gate_w, s_up_w,
      s_down_w, shared_gate.reshape(1, 1))


def _combine_body(yg_hbm, row_hbm, pw_hbm, sout_hbm, out_hbm,
                  idx0, idx1, w0v, w1v, r0buf, r1buf, obuf, sem):
    wid = lax.axis_index("s") * NC + lax.axis_index("c")
    for c in range(TPT // 16):
        tb = wid * TPT + c * 16
        pltpu.sync_copy(row_hbm.at[pl.ds(tb, 16)], idx0)
        pltpu.sync_copy(row_hbm.at[pl.ds(S + tb, 16)], idx1)
        pltpu.sync_copy(pw_hbm.at[pl.ds(tb, 16)], w0v)
        pltpu.sync_copy(pw_hbm.at[pl.ds(S + tb, 16)], w1v)
        pltpu.async_copy(yg_hbm.at[idx0], r0buf, sem).wait()
        pltpu.async_copy(yg_hbm.at[idx1], r1buf, sem).wait()
        pltpu.sync_copy(sout_hbm.at[pl.ds(tb, 16)], obuf)
        w0vec = w0v[...]
        w1vec = w1v[...]
        for j in range(16):
            w0s = w0vec[j]
            w1s = w1vec[j]

            def cb(ci, _, j=j, w0s=w0s, w1s=w1s):
                sl = pl.ds(ci * 16, 16)
                obuf[j, sl] = (obuf[j, sl] + w0s * r0buf[j, sl]
                               + w1s * r1buf[j, sl])
                return 0

            lax.fori_loop(0, H // 16, cb, 0)
        pltpu.sync_copy(obuf, out_hbm.at[pl.ds(tb, 16)])


def _combine(yg, pair_row, pw_flat, sout):
    mesh = plsc.VectorSubcoreMesh(core_axis_name="c", subcore_axis_name="s")
    f = pl.kernel(
        _combine_body,
        out_type=jax.ShapeDtypeStruct((S, H), jnp.float32),
        mesh=mesh,
        scratch_types=[
            pltpu.VMEM((16,), jnp.int32),
            pltpu.VMEM((16,), jnp.int32),
            pltpu.VMEM((16,), jnp.float32),
            pltpu.VMEM((16,), jnp.float32),
            pltpu.VMEM((16, H), jnp.float32),
            pltpu.VMEM((16, H), jnp.float32),
            pltpu.VMEM((16, H), jnp.float32),
            pltpu.SemaphoreType.DMA,
        ],
        compiler_params=pltpu.CompilerParams(needs_layout_passes=False),
    )
    return f(yg, pair_row, pw_flat, sout)


def kernel(hidden_states, ln_g, ln_b, router_w, expert_bias, gate_w, up_w,
           down_w, s_ln_g, s_ln_b, s_gate_w, s_up_w, s_down_w, shared_gate):
    B, S_, H_ = hidden_states.shape
    x = hidden_states.reshape(-1, H_)
    pair_eid, pair_w, zpart = _router(x, ln_g, ln_b, router_w, expert_bias)
    xg, pair_row, sched = _dispatch(x, pair_eid.reshape(NP))
    sout = _shared(x, s_ln_g, s_ln_b, s_gate_w, s_up_w, s_down_w,
                   shared_gate)
    yg = _gffn(sched, xg, gate_w, up_w, down_w)
    final = _combine(yg, pair_row, pair_w.reshape(NP), sout)
    z_loss = jnp.sum(zpart) / S_ * 0.0001
    return (final.reshape(B, S_, H_), z_loss)


# speedup vs baseline: 1.0125x; 1.0125x over previous
import functools

import jax
import jax.numpy as jnp
from jax import lax
from jax.experimental import pallas as pl
from jax.experimental.pallas import tpu as pltpu
from jax.experimental.pallas import tpu_sc as plsc

EPS = 1e-05
S = 2048
H = 2048
I = 4096
E = 8
K = 2
NP = K * S

TBR = 256
GR = S // TBR

TB = 256
NBLK = 23
CAP = NBLK * TB
NBCAP = 8
ICF = 512
NIF = I // ICF
NSTEPS = 192

TSUB = 512
ICS = 512
NIS = I // ICS

NC = 2
NS = 16
NW = NC * NS
PPT = NP // NW
TPT = S // NW


def _router_body(x_ref, g_ref, b_ref, rw_ref, bias_ref,
                 eid_ref, pw_ref, z_ref):
    b = pl.program_id(0)
    x = x_ref[...]
    m = jnp.mean(x, axis=1, keepdims=True)
    v = jnp.mean((x - m) ** 2, axis=1, keepdims=True)
    xn = (x - m) / jnp.sqrt(v + 1e-05) * g_ref[...] + b_ref[...]
    logits = jax.lax.dot_general(xn, rw_ref[...],
                                 (((1,), (1,)), ((), ())),
                                 preferred_element_type=jnp.float32)
    logits = logits + bias_ref[...]
    lmax = jnp.max(logits, axis=1, keepdims=True)
    ex = jnp.exp(logits - lmax)
    sex = jnp.sum(ex, axis=1, keepdims=True)
    lse = lmax + jnp.log(sex)
    p = ex / sex

    iota = lax.broadcasted_iota(jnp.int32, (TBR, E), 1)
    m1 = jnp.max(p, axis=1, keepdims=True)
    i1 = jnp.min(jnp.where(p == m1, iota, E), axis=1, keepdims=True)
    p2 = jnp.where(iota == i1, -jnp.inf, p)
    m2 = jnp.max(p2, axis=1, keepdims=True)
    i2 = jnp.min(jnp.where(p2 == m2, iota, E), axis=1, keepdims=True)
    ssum = jnp.clip(m1 + m2, EPS, None)
    w1 = m1 / ssum
    w2 = m2 / ssum
    eid_ref[0, pl.ds(b * TBR, TBR)] = i1[:, 0]
    eid_ref[1, pl.ds(b * TBR, TBR)] = i2[:, 0]
    pw_ref[0, pl.ds(b * TBR, TBR)] = w1[:, 0]
    pw_ref[1, pl.ds(b * TBR, TBR)] = w2[:, 0]
    zrow = jnp.sum(lse * lse)
    lane = lax.broadcasted_iota(jnp.int32, (1, 128), 1)
    z_ref[pl.ds(b, 1), :] = jnp.where(lane == 0, zrow, 0.0)


def _router(x, ln_g, ln_b, router_w, expert_bias):
    return pl.pallas_call(
        _router_body,
        grid=(GR,),
        in_specs=[
            pl.BlockSpec((TBR, H), lambda b: (b, 0)),
            pl.BlockSpec((1, H), lambda b: (0, 0)),
            pl.BlockSpec((1, H), lambda b: (0, 0)),
            pl.BlockSpec((E, H), lambda b: (0, 0)),
            pl.BlockSpec((1, E), lambda b: (0, 0)),
        ],
        out_specs=[
            pl.BlockSpec((K, S), lambda b: (0, 0)),
            pl.BlockSpec((K, S), lambda b: (0, 0)),
            pl.BlockSpec((GR, 128), lambda b: (0, 0)),
        ],
        out_shape=[
            jax.ShapeDtypeStruct((K, S), jnp.int32),
            jax.ShapeDtypeStruct((K, S), jnp.float32),
            jax.ShapeDtypeStruct((GR, 128), jnp.float32),
        ],
    )(x, ln_g.reshape(1, H), ln_b.reshape(1, H), router_w,
      expert_bias.reshape(1, E))


def _dispatch_body(x_hbm, eid_hbm, xg_hbm, row_hbm, sched_hbm,
                   eid_v, pos_v, schedv,
                   tok0, tok1, tok2, tok3, dst0, dst1, dst2, dst3,
                   xbuf, sem):
    wid = lax.axis_index("s") * NC + lax.axis_index("c")
    iota16 = lax.iota(jnp.int32, 16)
    pltpu.sync_copy(eid_hbm, eid_v)

    def scan_body(k, cnt):
        ev = eid_v[pl.ds(k * 16, 16)]
        base = cnt.at[ev].get(mode="promise_in_bounds")
        rank = jnp.zeros((16,), jnp.int32)
        newcnt = cnt
        for e in range(E):
            m = ev == e
            inc = jnp.where(m, 1, 0).astype(jnp.int32)
            cs = jnp.cumsum(inc)
            rank = rank + jnp.where(m, cs, 0)
            tot = jnp.sum(inc)
            newcnt = newcnt + jnp.where(iota16 == e, tot, 0)
        pos_v[pl.ds(k * 16, 16)] = base + rank - 1
        return newcnt

    cnt = lax.fori_loop(0, NP // 16, scan_body,
                        jnp.zeros((16,), jnp.int32))
    aligned = ((cnt + (TB - 1)) // TB) * TB
    incl = jnp.cumsum(aligned)
    start = incl - aligned

    pbase = wid * PPT
    tok = (tok0, tok1, tok2, tok3)
    dst = (dst0, dst1, dst2, dst3)
    for c in range(4):
        for hh in range(2):
            off = pbase + c * 32 + hh * 16
            ev = eid_v[pl.ds(off, 16)]
            posv = pos_v[pl.ds(off, 16)]
            destv = start.at[ev].get(mode="promise_in_bounds") + posv
            dst[c][pl.ds(hh * 16, 16)] = destv
            pv = off + iota16
            tok[c][pl.ds(hh * 16, 16)] = jnp.bitwise_and(pv, S - 1)
        pltpu.async_copy(x_hbm.at[tok[c]], xbuf, sem).wait()
        pltpu.async_copy(xbuf, xg_hbm.at[dst[c]], sem).wait()
        pltpu.sync_copy(dst[c], row_hbm.at[pl.ds(pbase + c * 32, 32)])

    @pl.when(wid == 0)
    def _():
        nb_vec = aligned // TB
        bstart_vec = start // TB
        e_of_g = iota16 // 2
        sg_of_g = jnp.bitwise_and(iota16, 1)
        nb_g = nb_vec.at[e_of_g].get(mode="promise_in_bounds")
        bb_g = (bstart_vec.at[e_of_g].get(mode="promise_in_bounds")
                + sg_of_g * NBCAP)
        nbs_g = jnp.clip(nb_g - sg_of_g * NBCAP, 0, NBCAP)
        steps_g = NIF * nbs_g
        cum_g = jnp.cumsum(steps_g)
        segstart_g = cum_g - steps_g
        total = cum_g[15]
        laste = jnp.int32(0)
        lastb = jnp.int32(0)
        lastblk = jnp.int32(0)
        for g in range(16):
            ne = nbs_g[g] > 0
            laste = jnp.where(ne, jnp.int32(g // 2), laste)
            lastb = jnp.where(ne, nbs_g[g] - 1, lastb)
            lastblk = jnp.where(ne, bb_g[g] + nbs_g[g] - 1, lastblk)
        def colbody(ci, carry):
            sv = ci * 16 + iota16
            e_col = jnp.zeros((16,), jnp.int32)
            i_col = jnp.zeros((16,), jnp.int32)
            b_col = jnp.zeros((16,), jnp.int32)
            blk_col = jnp.zeros((16,), jnp.int32)
            for g in range(16):
                ss = segstart_g[g]
                nbs_s = jnp.maximum(nbs_g[g], 1)
                m = (sv >= ss) & (sv < ss + NIF * nbs_g[g])
                k = sv - ss
                i_val = k // nbs_s
                b_val = k - i_val * nbs_s
                e_col = jnp.where(m, jnp.int32(g // 2), e_col)
                i_col = jnp.where(m, i_val, i_col)
                b_col = jnp.where(m, b_val, b_col)
                blk_col = jnp.where(m, bb_g[g] + b_val, blk_col)
            valid = sv < total
            sl = pl.ds(ci * 16, 16)
            schedv[0, sl] = jnp.where(valid, e_col, laste)
            schedv[1, sl] = jnp.where(valid, i_col, NIF - 1)
            schedv[2, sl] = jnp.where(valid, b_col, lastb)
            schedv[3, sl] = jnp.where(valid, blk_col, lastblk)
            schedv[4, sl] = jnp.where(valid, 1, 0)
            return carry

        lax.fori_loop(0, NSTEPS // 16, colbody, 0)
        pltpu.sync_copy(schedv, sched_hbm)


def _dispatch(x, eid_flat):
    mesh = plsc.VectorSubcoreMesh(core_axis_name="c", subcore_axis_name="s")
    f = pl.kernel(
        _dispatch_body,
        out_type=[
            jax.ShapeDtypeStruct((CAP, H), jnp.float32),
            jax.ShapeDtypeStruct((NP,), jnp.int32),
            jax.ShapeDtypeStruct((5, NSTEPS), jnp.int32),
        ],
        mesh=mesh,
        scratch_types=[
            pltpu.VMEM((NP,), jnp.int32),
            pltpu.VMEM((NP,), jnp.int32),
            pltpu.VMEM((5, NSTEPS), jnp.int32),
            pltpu.VMEM((32,), jnp.int32),
            pltpu.VMEM((32,), jnp.int32),
            pltpu.VMEM((32,), jnp.int32),
            pltpu.VMEM((32,), jnp.int32),
            pltpu.VMEM((32,), jnp.int32),
            pltpu.VMEM((32,), jnp.int32),
            pltpu.VMEM((32,), jnp.int32),
            pltpu.VMEM((32,), jnp.int32),
            pltpu.VMEM((32, H), jnp.float32),
            pltpu.SemaphoreType.DMA,
        ],
        compiler_params=pltpu.CompilerParams(needs_layout_passes=False),
    )
    return f(x, eid_flat)


def _ffn_chunk(x, gw, uw, dw):
    g = jax.lax.dot_general(x, gw, (((1,), (1,)), ((), ())),
                            preferred_element_type=jnp.float32)
    g = g * jax.nn.sigmoid(g)
    u = jax.lax.dot_general(x, uw, (((1,), (1,)), ((), ())),
                            preferred_element_type=jnp.float32)
    h = g * u
    return jax.lax.dot_general(h, dw, (((1,), (1,)), ((), ())),
                               preferred_element_type=jnp.float32)


def _gffn_body(sched_ref, xg_ref, gw_ref, uw_ref, dw_ref, yg_ref,
               acc, xsc, sem, sem2):
    s = pl.program_id(0)
    i = sched_ref[1, s]
    ba = sched_ref[2, s]
    blk = sched_ref[3, s]
    valid = sched_ref[4, s]

    @pl.when(valid == 1)
    def _():
        row = pl.multiple_of(ba * TB, TB)
        grow = pl.multiple_of(blk * TB, TB)

        @pl.when(i == 0)
        def _():
            cp = pltpu.make_async_copy(
                xg_ref.at[pl.ds(grow, TB), :],
                xsc.at[pl.ds(row, TB), :], sem2)
            cp.start()
            cp.wait()

        y = _ffn_chunk(xsc[pl.ds(row, TB), :], gw_ref[0], uw_ref[0],
                       dw_ref[0])

        @pl.when(i == 0)
        def _():
            acc[pl.ds(row, TB), :] = y

        @pl.when(i > 0)
        def _():
            acc[pl.ds(row, TB), :] = acc[pl.ds(row, TB), :] + y

        @pl.when(i == NIF - 1)
        def _():
            cp = pltpu.make_async_copy(
                acc.at[pl.ds(row, TB), :],
                yg_ref.at[pl.ds(grow, TB), :], sem)
            cp.start()
            cp.wait()


def _gffn(sched, xg, gate_w, up_w, down_w):
    grid_spec = pltpu.PrefetchScalarGridSpec(
        num_scalar_prefetch=1,
        grid=(NSTEPS,),
        in_specs=[
            pl.BlockSpec(memory_space=pl.ANY),
            pl.BlockSpec((1, ICF, H), lambda s, sc: (sc[0, s], sc[1, s], 0)),
            pl.BlockSpec((1, ICF, H), lambda s, sc: (sc[0, s], sc[1, s], 0)),
            pl.BlockSpec((1, H, ICF), lambda s, sc: (sc[0, s], 0, sc[1, s])),
        ],
        out_specs=pl.BlockSpec(memory_space=pl.ANY),
        scratch_shapes=[
            pltpu.VMEM((NBCAP * TB, H), jnp.float32),
            pltpu.VMEM((NBCAP * TB, H), jnp.float32),
            pltpu.SemaphoreType.DMA,
            pltpu.SemaphoreType.DMA,
        ],
    )
    return pl.pallas_call(
        _gffn_body,
        grid_spec=grid_spec,
        out_shape=jax.ShapeDtypeStruct((CAP, H), jnp.float32),
        compiler_params=pltpu.CompilerParams(
            vmem_limit_bytes=60 * 1024 * 1024),
    )(sched, xg, gate_w, up_w, down_w)


def _shared_body(x_ref, g_ref, b_ref, gw_ref, uw_ref, dw_ref, sg_ref,
                 out_ref):
    i = pl.program_id(0)
    for ts in range(S // TSUB):
        sl = pl.ds(ts * TSUB, TSUB)
        x = x_ref[sl, :]
        m = jnp.mean(x, axis=1, keepdims=True)
        v = jnp.mean((x - m) ** 2, axis=1, keepdims=True)
        sx = (x - m) / jnp.sqrt(v + 1e-05) * g_ref[...] + b_ref[...]
        y = _ffn_chunk(sx, gw_ref[...], uw_ref[...], dw_ref[...])

        @pl.when(i == 0)
        def _():
            out_ref[sl, :] = y

        @pl.when(i > 0)
        def _():
            out_ref[sl, :] = out_ref[sl, :] + y

        @pl.when(i == NIS - 1)
        def _():
            sig = jax.nn.sigmoid(sg_ref[0, 0])
            out_ref[sl, :] = out_ref[sl, :] * sig


def _shared(x, s_ln_g, s_ln_b, s_gate_w, s_up_w, s_down_w, shared_gate):
    return pl.pallas_call(
        _shared_body,
        grid=(NIS,),
        in_specs=[
            pl.BlockSpec((S, H), lambda i: (0, 0)),
            pl.BlockSpec((1, H), lambda i: (0, 0)),
            pl.BlockSpec((1, H), lambda i: (0, 0)),
            pl.BlockSpec((ICS, H), lambda i: (i, 0)),
            pl.BlockSpec((ICS, H), lambda i: (i, 0)),
            pl.BlockSpec((H, ICS), lambda i: (0, i)),
            pl.BlockSpec((1, 1), lambda i: (0, 0)),
        ],
        out_specs=pl.BlockSpec((S, H), lambda i: (0, 0)),
        out_shape=jax.ShapeDtypeStruct((S, H), jnp.float32),
        compiler_params=pltpu.CompilerParams(
            vmem_limit_bytes=62 * 1024 * 1024),
    )(x, s_ln_g.reshape(1, H), s_ln_b.reshape(1, H), s_gate_w, s_up_w,
      s_down_w, shared_gate.reshape(1, 1))


def _combine_body(yg_hbm, row_hbm, pw_hbm, sout_hbm, out_hbm,
                  idx0, idx1, w0v, w1v, r0buf, r1buf, obuf, sem):
    wid = lax.axis_index("s") * NC + lax.axis_index("c")
    for c in range(TPT // 16):
        tb = wid * TPT + c * 16
        pltpu.sync_copy(row_hbm.at[pl.ds(tb, 16)], idx0)
        pltpu.sync_copy(row_hbm.at[pl.ds(S + tb, 16)], idx1)
        pltpu.sync_copy(pw_hbm.at[pl.ds(tb, 16)], w0v)
        pltpu.sync_copy(pw_hbm.at[pl.ds(S + tb, 16)], w1v)
        pltpu.async_copy(yg_hbm.at[idx0], r0buf, sem).wait()
        pltpu.async_copy(yg_hbm.at[idx1], r1buf, sem).wait()
        pltpu.sync_copy(sout_hbm.at[pl.ds(tb, 16)], obuf)
        w0vec = w0v[...]
        w1vec = w1v[...]
        for j in range(16):
            w0s = w0vec[j]
            w1s = w1vec[j]

            def cb(ci, _, j=j, w0s=w0s, w1s=w1s):
                sl = pl.ds(ci * 16, 16)
                obuf[j, sl] = (obuf[j, sl] + w0s * r0buf[j, sl]
                               + w1s * r1buf[j, sl])
                return 0

            lax.fori_loop(0, H // 16, cb, 0)
        pltpu.sync_copy(obuf, out_hbm.at[pl.ds(tb, 16)])


def _combine(yg, pair_row, pw_flat, sout):
    mesh = plsc.VectorSubcoreMesh(core_axis_name="c", subcore_axis_name="s")
    f = pl.kernel(
        _combine_body,
        out_type=jax.ShapeDtypeStruct((S, H), jnp.float32),
        mesh=mesh,
        scratch_types=[
            pltpu.VMEM((16,), jnp.int32),
            pltpu.VMEM((16,), jnp.int32),
            pltpu.VMEM((16,), jnp.float32),
            pltpu.VMEM((16,), jnp.float32),
            pltpu.VMEM((16, H), jnp.float32),
            pltpu.VMEM((16, H), jnp.float32),
            pltpu.VMEM((16, H), jnp.float32),
            pltpu.SemaphoreType.DMA,
        ],
        compiler_params=pltpu.CompilerParams(needs_layout_passes=False),
    )
    return f(yg, pair_row, pw_flat, sout)


def kernel(hidden_states, ln_g, ln_b, router_w, expert_bias, gate_w, up_w,
           down_w, s_ln_g, s_ln_b, s_gate_w, s_up_w, s_down_w, shared_gate):
    B, S_, H_ = hidden_states.shape
    x = hidden_states.reshape(-1, H_)
    pair_eid, pair_w, zpart = _router(x, ln_g, ln_b, router_w, expert_bias)
    xg, pair_row, sched = _dispatch(x, pair_eid.reshape(NP))
    sout = _shared(x, s_ln_g, s_ln_b, s_gate_w, s_up_w, s_down_w,
                   shared_gate)
    yg = _gffn(sched, xg, gate_w, up_w, down_w)
    final = _combine(yg, pair_row, pair_w.reshape(NP), sout)
    z_loss = jnp.sum(zpart) / S_ * 0.0001
    return (final.reshape(B, S_, H_), z_loss)


# speedup vs baseline: 1.0395x; 1.0267x over previous
import functools

import jax
import jax.numpy as jnp
from jax import lax
from jax.experimental import pallas as pl
from jax.experimental.pallas import tpu as pltpu
from jax.experimental.pallas import tpu_sc as plsc

EPS = 1e-05
S = 2048
H = 2048
I = 4096
E = 8
K = 2
NP = K * S

TBR = 256
GR = S // TBR

TB = 256
NBLK = 23
CAP = NBLK * TB
NBCAP = 8
ICF = 512
NIF = I // ICF
NSTEPS = 192

TSUB = 512
ICS = 512
NIS = I // ICS

NC = 2
NS = 16
NW = NC * NS
PPT = NP // NW
TPT = S // NW


def _router_body(x_ref, g_ref, b_ref, rw_ref, bias_ref,
                 eid_ref, pw_ref, z_ref):
    b = pl.program_id(0)
    x = x_ref[...]
    m = jnp.mean(x, axis=1, keepdims=True)
    v = jnp.mean((x - m) ** 2, axis=1, keepdims=True)
    xn = (x - m) / jnp.sqrt(v + 1e-05) * g_ref[...] + b_ref[...]
    logits = jax.lax.dot_general(xn, rw_ref[...],
                                 (((1,), (1,)), ((), ())),
                                 preferred_element_type=jnp.float32)
    logits = logits + bias_ref[...]
    lmax = jnp.max(logits, axis=1, keepdims=True)
    ex = jnp.exp(logits - lmax)
    sex = jnp.sum(ex, axis=1, keepdims=True)
    lse = lmax + jnp.log(sex)
    p = ex / sex

    iota = lax.broadcasted_iota(jnp.int32, (TBR, E), 1)
    m1 = jnp.max(p, axis=1, keepdims=True)
    i1 = jnp.min(jnp.where(p == m1, iota, E), axis=1, keepdims=True)
    p2 = jnp.where(iota == i1, -jnp.inf, p)
    m2 = jnp.max(p2, axis=1, keepdims=True)
    i2 = jnp.min(jnp.where(p2 == m2, iota, E), axis=1, keepdims=True)
    ssum = jnp.clip(m1 + m2, EPS, None)
    w1 = m1 / ssum
    w2 = m2 / ssum
    eid_ref[0, pl.ds(b * TBR, TBR)] = i1[:, 0]
    eid_ref[1, pl.ds(b * TBR, TBR)] = i2[:, 0]
    pw_ref[0, pl.ds(b * TBR, TBR)] = w1[:, 0]
    pw_ref[1, pl.ds(b * TBR, TBR)] = w2[:, 0]
    zrow = jnp.sum(lse * lse)
    lane = lax.broadcasted_iota(jnp.int32, (1, 128), 1)
    z_ref[pl.ds(b, 1), :] = jnp.where(lane == 0, zrow, 0.0)


def _router(x, ln_g, ln_b, router_w, expert_bias):
    return pl.pallas_call(
        _router_body,
        grid=(GR,),
        in_specs=[
            pl.BlockSpec((TBR, H), lambda b: (b, 0)),
            pl.BlockSpec((1, H), lambda b: (0, 0)),
            pl.BlockSpec((1, H), lambda b: (0, 0)),
            pl.BlockSpec((E, H), lambda b: (0, 0)),
            pl.BlockSpec((1, E), lambda b: (0, 0)),
        ],
        out_specs=[
            pl.BlockSpec((K, S), lambda b: (0, 0)),
            pl.BlockSpec((K, S), lambda b: (0, 0)),
            pl.BlockSpec((GR, 128), lambda b: (0, 0)),
        ],
        out_shape=[
            jax.ShapeDtypeStruct((K, S), jnp.int32),
            jax.ShapeDtypeStruct((K, S), jnp.float32),
            jax.ShapeDtypeStruct((GR, 128), jnp.float32),
        ],
    )(x, ln_g.reshape(1, H), ln_b.reshape(1, H), router_w,
      expert_bias.reshape(1, E))


def _dispatch_body(x_hbm, eid_hbm, xg_hbm, row_hbm, sched_hbm,
                   eid_v, pos_v, schedv,
                   tok0, tok1, tok2, tok3, dst0, dst1, dst2, dst3,
                   xbuf, sem):
    wid = lax.axis_index("s") * NC + lax.axis_index("c")
    iota16 = lax.iota(jnp.int32, 16)
    pltpu.sync_copy(eid_hbm, eid_v)

    def scan_body(k, cnt):
        ev = eid_v[pl.ds(k * 16, 16)]
        base = cnt.at[ev].get(mode="promise_in_bounds")
        rank = jnp.zeros((16,), jnp.int32)
        newcnt = cnt
        for e in range(E):
            m = ev == e
            inc = jnp.where(m, 1, 0).astype(jnp.int32)
            cs = jnp.cumsum(inc)
            rank = rank + jnp.where(m, cs, 0)
            tot = jnp.sum(inc)
            newcnt = newcnt + jnp.where(iota16 == e, tot, 0)
        pos_v[pl.ds(k * 16, 16)] = base + rank - 1
        return newcnt

    cnt = lax.fori_loop(0, NP // 16, scan_body,
                        jnp.zeros((16,), jnp.int32))
    aligned = ((cnt + (TB - 1)) // TB) * TB
    incl = jnp.cumsum(aligned)
    start = incl - aligned

    pbase = wid * PPT
    tok = (tok0, tok1, tok2, tok3)
    dst = (dst0, dst1, dst2, dst3)
    for c in range(4):
        for hh in range(2):
            off = pbase + c * 32 + hh * 16
            ev = eid_v[pl.ds(off, 16)]
            posv = pos_v[pl.ds(off, 16)]
            destv = start.at[ev].get(mode="promise_in_bounds") + posv
            dst[c][pl.ds(hh * 16, 16)] = destv
            pv = off + iota16
            tok[c][pl.ds(hh * 16, 16)] = jnp.bitwise_and(pv, S - 1)
        pltpu.async_copy(x_hbm.at[tok[c]], xbuf, sem).wait()
        pltpu.async_copy(xbuf, xg_hbm.at[dst[c]], sem).wait()
        pltpu.sync_copy(dst[c], row_hbm.at[pl.ds(pbase + c * 32, 32)])

    @pl.when(wid == 0)
    def _():
        nb_vec = aligned // TB
        bstart_vec = start // TB
        e_of_g = iota16 // 2
        sg_of_g = jnp.bitwise_and(iota16, 1)
        nb_g = nb_vec.at[e_of_g].get(mode="promise_in_bounds")
        bb_g = (bstart_vec.at[e_of_g].get(mode="promise_in_bounds")
                + sg_of_g * NBCAP)
        nbs_g = jnp.clip(nb_g - sg_of_g * NBCAP, 0, NBCAP)
        steps_g = NIF * nbs_g
        cum_g = jnp.cumsum(steps_g)
        segstart_g = cum_g - steps_g
        total = cum_g[15]
        laste = jnp.int32(0)
        lastb = jnp.int32(0)
        lastblk = jnp.int32(0)
        for g in range(16):
            ne = nbs_g[g] > 0
            laste = jnp.where(ne, jnp.int32(g // 2), laste)
            lastb = jnp.where(ne, nbs_g[g] - 1, lastb)
            lastblk = jnp.where(ne, bb_g[g] + nbs_g[g] - 1, lastblk)
        def colbody(ci, carry):
            sv = ci * 16 + iota16
            e_col = jnp.zeros((16,), jnp.int32)
            i_col = jnp.zeros((16,), jnp.int32)
            b_col = jnp.zeros((16,), jnp.int32)
            blk_col = jnp.zeros((16,), jnp.int32)
            for g in range(16):
                ss = segstart_g[g]
                nbs_s = jnp.maximum(nbs_g[g], 1)
                m = (sv >= ss) & (sv < ss + NIF * nbs_g[g])
                k = sv - ss
                i_val = k // nbs_s
                b_val = k - i_val * nbs_s
                e_col = jnp.where(m, jnp.int32(g // 2), e_col)
                i_col = jnp.where(m, i_val, i_col)
                b_col = jnp.where(m, b_val, b_col)
                blk_col = jnp.where(m, bb_g[g] + b_val, blk_col)
            valid = sv < total
            sl = pl.ds(ci * 16, 16)
            schedv[0, sl] = jnp.where(valid, e_col, laste)
            schedv[1, sl] = jnp.where(valid, i_col, NIF - 1)
            schedv[2, sl] = jnp.where(valid, b_col, lastb)
            schedv[3, sl] = jnp.where(valid, blk_col, lastblk)
            schedv[4, sl] = jnp.where(valid, 1, 0)
            return carry

        lax.fori_loop(0, NSTEPS // 16, colbody, 0)
        pltpu.sync_copy(schedv, sched_hbm)


def _dispatch(x, eid_flat):
    mesh = plsc.VectorSubcoreMesh(core_axis_name="c", subcore_axis_name="s")
    f = pl.kernel(
        _dispatch_body,
        out_type=[
            jax.ShapeDtypeStruct((CAP, H), jnp.float32),
            jax.ShapeDtypeStruct((NP,), jnp.int32),
            jax.ShapeDtypeStruct((5, NSTEPS), jnp.int32),
        ],
        mesh=mesh,
        scratch_types=[
            pltpu.VMEM((NP,), jnp.int32),
            pltpu.VMEM((NP,), jnp.int32),
            pltpu.VMEM((5, NSTEPS), jnp.int32),
            pltpu.VMEM((32,), jnp.int32),
            pltpu.VMEM((32,), jnp.int32),
            pltpu.VMEM((32,), jnp.int32),
            pltpu.VMEM((32,), jnp.int32),
            pltpu.VMEM((32,), jnp.int32),
            pltpu.VMEM((32,), jnp.int32),
            pltpu.VMEM((32,), jnp.int32),
            pltpu.VMEM((32,), jnp.int32),
            pltpu.VMEM((32, H), jnp.float32),
            pltpu.SemaphoreType.DMA,
        ],
        compiler_params=pltpu.CompilerParams(needs_layout_passes=False),
    )
    return f(x, eid_flat)


def _ffn_chunk(x, gw, uw, dw):
    g = jax.lax.dot_general(x, gw, (((1,), (1,)), ((), ())),
                            preferred_element_type=jnp.float32)
    g = g * jax.nn.sigmoid(g)
    u = jax.lax.dot_general(x, uw, (((1,), (1,)), ((), ())),
                            preferred_element_type=jnp.float32)
    h = g * u
    return jax.lax.dot_general(h, dw, (((1,), (1,)), ((), ())),
                               preferred_element_type=jnp.float32)


def _ffn_chunk_bf16(x, gw, uw, dw):
    xb = x.astype(jnp.bfloat16)
    g = jax.lax.dot_general(xb, gw.astype(jnp.bfloat16),
                            (((1,), (1,)), ((), ())),
                            preferred_element_type=jnp.float32)
    g = g * jax.nn.sigmoid(g)
    u = jax.lax.dot_general(xb, uw.astype(jnp.bfloat16),
                            (((1,), (1,)), ((), ())),
                            preferred_element_type=jnp.float32)
    h = g * u
    return jax.lax.dot_general(h.astype(jnp.bfloat16),
                               dw.astype(jnp.bfloat16),
                               (((1,), (1,)), ((), ())),
                               preferred_element_type=jnp.float32)


def _gffn_body(sched_ref, xg_ref, gw_ref, uw_ref, dw_ref, yg_ref,
               acc, xsc, sem, sem2):
    s = pl.program_id(0)
    i = sched_ref[1, s]
    ba = sched_ref[2, s]
    blk = sched_ref[3, s]
    valid = sched_ref[4, s]

    @pl.when(valid == 1)
    def _():
        row = pl.multiple_of(ba * TB, TB)
        grow = pl.multiple_of(blk * TB, TB)

        @pl.when(i == 0)
        def _():
            cp = pltpu.make_async_copy(
                xg_ref.at[pl.ds(grow, TB), :],
                xsc.at[pl.ds(row, TB), :], sem2)
            cp.start()
            cp.wait()

        y = _ffn_chunk(xsc[pl.ds(row, TB), :], gw_ref[0], uw_ref[0],
                       dw_ref[0])

        @pl.when(i == 0)
        def _():
            acc[pl.ds(row, TB), :] = y

        @pl.when(i > 0)
        def _():
            acc[pl.ds(row, TB), :] = acc[pl.ds(row, TB), :] + y

        @pl.when(i == NIF - 1)
        def _():
            cp = pltpu.make_async_copy(
                acc.at[pl.ds(row, TB), :],
                yg_ref.at[pl.ds(grow, TB), :], sem)
            cp.start()
            cp.wait()


def _gffn(sched, xg, gate_w, up_w, down_w):
    grid_spec = pltpu.PrefetchScalarGridSpec(
        num_scalar_prefetch=1,
        grid=(NSTEPS,),
        in_specs=[
            pl.BlockSpec(memory_space=pl.ANY),
            pl.BlockSpec((1, ICF, H), lambda s, sc: (sc[0, s], sc[1, s], 0)),
            pl.BlockSpec((1, ICF, H), lambda s, sc: (sc[0, s], sc[1, s], 0)),
            pl.BlockSpec((1, H, ICF), lambda s, sc: (sc[0, s], 0, sc[1, s])),
        ],
        out_specs=pl.BlockSpec(memory_space=pl.ANY),
        scratch_shapes=[
            pltpu.VMEM((NBCAP * TB, H), jnp.float32),
            pltpu.VMEM((NBCAP * TB, H), jnp.float32),
            pltpu.SemaphoreType.DMA,
            pltpu.SemaphoreType.DMA,
        ],
    )
    return pl.pallas_call(
        _gffn_body,
        grid_spec=grid_spec,
        out_shape=jax.ShapeDtypeStruct((CAP, H), jnp.float32),
        compiler_params=pltpu.CompilerParams(
            vmem_limit_bytes=60 * 1024 * 1024),
    )(sched, xg, gate_w, up_w, down_w)


def _shared_body(x_ref, g_ref, b_ref, gw_ref, uw_ref, dw_ref, sg_ref,
                 out_ref):
    i = pl.program_id(0)
    for ts in range(S // TSUB):
        sl = pl.ds(ts * TSUB, TSUB)
        x = x_ref[sl, :]
        m = jnp.mean(x, axis=1, keepdims=True)
        v = jnp.mean((x - m) ** 2, axis=1, keepdims=True)
        sx = (x - m) / jnp.sqrt(v + 1e-05) * g_ref[...] + b_ref[...]
        y = _ffn_chunk_bf16(sx, gw_ref[...], uw_ref[...], dw_ref[...])

        @pl.when(i == 0)
        def _():
            out_ref[sl, :] = y

        @pl.when(i > 0)
        def _():
            out_ref[sl, :] = out_ref[sl, :] + y

        @pl.when(i == NIS - 1)
        def _():
            sig = jax.nn.sigmoid(sg_ref[0, 0])
            out_ref[sl, :] = out_ref[sl, :] * sig


def _shared(x, s_ln_g, s_ln_b, s_gate_w, s_up_w, s_down_w, shared_gate):
    return pl.pallas_call(
        _shared_body,
        grid=(NIS,),
        in_specs=[
            pl.BlockSpec((S, H), lambda i: (0, 0)),
            pl.BlockSpec((1, H), lambda i: (0, 0)),
            pl.BlockSpec((1, H), lambda i: (0, 0)),
            pl.BlockSpec((ICS, H), lambda i: (i, 0)),
            pl.BlockSpec((ICS, H), lambda i: (i, 0)),
            pl.BlockSpec((H, ICS), lambda i: (0, i)),
            pl.BlockSpec((1, 1), lambda i: (0, 0)),
        ],
        out_specs=pl.BlockSpec((S, H), lambda i: (0, 0)),
        out_shape=jax.ShapeDtypeStruct((S, H), jnp.float32),
        compiler_params=pltpu.CompilerParams(
            vmem_limit_bytes=62 * 1024 * 1024),
    )(x, s_ln_g.reshape(1, H), s_ln_b.reshape(1, H), s_gate_w, s_up_w,
      s_down_w, shared_gate.reshape(1, 1))


CHT = 8
NCH = TPT // CHT


def _combine_body(yg_hbm, row_hbm, pw_hbm, sout_hbm, out_hbm,
                  idx0, idx1, w0v, w1v,
                  r0a, r1a, oba, r0b, r1b, obb, sema, semb, semo):
    wid = lax.axis_index("s") * NC + lax.axis_index("c")
    base = wid * TPT
    pltpu.sync_copy(row_hbm.at[pl.ds(base, TPT)], idx0)
    pltpu.sync_copy(row_hbm.at[pl.ds(S + base, TPT)], idx1)
    pltpu.sync_copy(pw_hbm.at[pl.ds(base, TPT)], w0v)
    pltpu.sync_copy(pw_hbm.at[pl.ds(S + base, TPT)], w1v)
    bufs = ((r0a, r1a, oba, sema), (r0b, r1b, obb, semb))

    def issue(c):
        r0, r1, ob, sem = bufs[c % 2]
        tb = base + c * CHT
        pltpu.async_copy(yg_hbm.at[idx0.at[pl.ds(c * CHT, CHT)]], r0, sem)
        pltpu.async_copy(yg_hbm.at[idx1.at[pl.ds(c * CHT, CHT)]], r1, sem)
        pltpu.async_copy(sout_hbm.at[pl.ds(tb, CHT)], ob, sem)

    issue(0)
    for c in range(NCH):
        r0, r1, ob, sem = bufs[c % 2]
        if c + 1 < NCH:
            issue(c + 1)
        pltpu.make_async_copy(yg_hbm.at[idx0.at[pl.ds(c * CHT, CHT)]],
                              r0, sem).wait()
        pltpu.make_async_copy(yg_hbm.at[idx1.at[pl.ds(c * CHT, CHT)]],
                              r1, sem).wait()
        pltpu.make_async_copy(sout_hbm.at[pl.ds(base + c * CHT, CHT)],
                              ob, sem).wait()
        for j in range(CHT):
            wsl = pl.ds((c * CHT + j) // 16 * 16, 16)
            w0s = w0v[wsl][(c * CHT + j) % 16]
            w1s = w1v[wsl][(c * CHT + j) % 16]

            def cb(ci, _, j=j, w0s=w0s, w1s=w1s, r0=r0, r1=r1, ob=ob):
                sl = pl.ds(ci * 16, 16)
                ob[j, sl] = (ob[j, sl] + w0s * r0[j, sl] + w1s * r1[j, sl])
                return 0

            lax.fori_loop(0, H // 16, cb, 0)
        pltpu.async_copy(ob, out_hbm.at[pl.ds(base + c * CHT, CHT)],
                         semo).wait()


def _combine(yg, pair_row, pw_flat, sout):
    mesh = plsc.VectorSubcoreMesh(core_axis_name="c", subcore_axis_name="s")
    f = pl.kernel(
        _combine_body,
        out_type=jax.ShapeDtypeStruct((S, H), jnp.float32),
        mesh=mesh,
        scratch_types=[
            pltpu.VMEM((TPT,), jnp.int32),
            pltpu.VMEM((TPT,), jnp.int32),
            pltpu.VMEM((TPT,), jnp.float32),
            pltpu.VMEM((TPT,), jnp.float32),
            pltpu.VMEM((CHT, H), jnp.float32),
            pltpu.VMEM((CHT, H), jnp.float32),
            pltpu.VMEM((CHT, H), jnp.float32),
            pltpu.VMEM((CHT, H), jnp.float32),
            pltpu.VMEM((CHT, H), jnp.float32),
            pltpu.VMEM((CHT, H), jnp.float32),
            pltpu.SemaphoreType.DMA,
            pltpu.SemaphoreType.DMA,
            pltpu.SemaphoreType.DMA,
        ],
        compiler_params=pltpu.CompilerParams(needs_layout_passes=False),
    )
    return f(yg, pair_row, pw_flat, sout)


def kernel(hidden_states, ln_g, ln_b, router_w, expert_bias, gate_w, up_w,
           down_w, s_ln_g, s_ln_b, s_gate_w, s_up_w, s_down_w, shared_gate):
    B, S_, H_ = hidden_states.shape
    x = hidden_states.reshape(-1, H_)
    pair_eid, pair_w, zpart = _router(x, ln_g, ln_b, router_w, expert_bias)
    xg, pair_row, sched = _dispatch(x, pair_eid.reshape(NP))
    sout = _shared(x, s_ln_g, s_ln_b, s_gate_w, s_up_w, s_down_w,
                   shared_gate)
    yg = _gffn(sched, xg, gate_w, up_w, down_w)
    final = _combine(yg, pair_row, pair_w.reshape(NP), sout)
    z_loss = jnp.sum(zpart) / S_ * 0.0001
    return (final.reshape(B, S_, H_), z_loss)


# speedup vs baseline: 1.0563x; 1.0162x over previous
import functools

import jax
import jax.numpy as jnp
from jax import lax
from jax.experimental import pallas as pl
from jax.experimental.pallas import tpu as pltpu
from jax.experimental.pallas import tpu_sc as plsc

EPS = 1e-05
S = 2048
H = 2048
I = 4096
E = 8
K = 2
NP = K * S

TBR = 256
GR = S // TBR

TB = 256
NBLK = 23
CAP = NBLK * TB
NBCAP = 8
ICF = 512
NIF = I // ICF
NSTEPS = 192

TSUB = 512
ICS = 512
NIS = I // ICS

NC = 2
NS = 16
NW = NC * NS
PPT = NP // NW
TPT = S // NW


def _router_body(x_ref, g_ref, b_ref, sg_ref, sb_ref, rw_ref, bias_ref,
                 eid_ref, pw_ref, z_ref, sx_ref):
    b = pl.program_id(0)
    x = x_ref[...]
    m = jnp.mean(x, axis=1, keepdims=True)
    v = jnp.mean((x - m) ** 2, axis=1, keepdims=True)
    xc = (x - m) / jnp.sqrt(v + 1e-05)
    xn = xc * g_ref[...] + b_ref[...]
    sx_ref[...] = xc * sg_ref[...] + sb_ref[...]
    logits = jax.lax.dot_general(xn, rw_ref[...],
                                 (((1,), (1,)), ((), ())),
                                 preferred_element_type=jnp.float32)
    logits = logits + bias_ref[...]
    lmax = jnp.max(logits, axis=1, keepdims=True)
    ex = jnp.exp(logits - lmax)
    sex = jnp.sum(ex, axis=1, keepdims=True)
    lse = lmax + jnp.log(sex)
    p = ex / sex

    iota = lax.broadcasted_iota(jnp.int32, (TBR, E), 1)
    m1 = jnp.max(p, axis=1, keepdims=True)
    i1 = jnp.min(jnp.where(p == m1, iota, E), axis=1, keepdims=True)
    p2 = jnp.where(iota == i1, -jnp.inf, p)
    m2 = jnp.max(p2, axis=1, keepdims=True)
    i2 = jnp.min(jnp.where(p2 == m2, iota, E), axis=1, keepdims=True)
    ssum = jnp.clip(m1 + m2, EPS, None)
    w1 = m1 / ssum
    w2 = m2 / ssum
    eid_ref[0, pl.ds(b * TBR, TBR)] = i1[:, 0]
    eid_ref[1, pl.ds(b * TBR, TBR)] = i2[:, 0]
    pw_ref[0, pl.ds(b * TBR, TBR)] = w1[:, 0]
    pw_ref[1, pl.ds(b * TBR, TBR)] = w2[:, 0]
    zrow = jnp.sum(lse * lse)
    lane = lax.broadcasted_iota(jnp.int32, (1, 128), 1)
    z_ref[pl.ds(b, 1), :] = jnp.where(lane == 0, zrow, 0.0)


def _router(x, ln_g, ln_b, s_ln_g, s_ln_b, router_w, expert_bias):
    return pl.pallas_call(
        _router_body,
        grid=(GR,),
        in_specs=[
            pl.BlockSpec((TBR, H), lambda b: (b, 0)),
            pl.BlockSpec((1, H), lambda b: (0, 0)),
            pl.BlockSpec((1, H), lambda b: (0, 0)),
            pl.BlockSpec((1, H), lambda b: (0, 0)),
            pl.BlockSpec((1, H), lambda b: (0, 0)),
            pl.BlockSpec((E, H), lambda b: (0, 0)),
            pl.BlockSpec((1, E), lambda b: (0, 0)),
        ],
        out_specs=[
            pl.BlockSpec((K, S), lambda b: (0, 0)),
            pl.BlockSpec((K, S), lambda b: (0, 0)),
            pl.BlockSpec((GR, 128), lambda b: (0, 0)),
            pl.BlockSpec((TBR, H), lambda b: (b, 0)),
        ],
        out_shape=[
            jax.ShapeDtypeStruct((K, S), jnp.int32),
            jax.ShapeDtypeStruct((K, S), jnp.float32),
            jax.ShapeDtypeStruct((GR, 128), jnp.float32),
            jax.ShapeDtypeStruct((S, H), jnp.float32),
        ],
    )(x, ln_g.reshape(1, H), ln_b.reshape(1, H), s_ln_g.reshape(1, H),
      s_ln_b.reshape(1, H), router_w, expert_bias.reshape(1, E))


def _dispatch_body(x_hbm, eid_hbm, xg_hbm, row_hbm, sched_hbm,
                   eid_v, pos_v, schedv,
                   tok0, tok1, tok2, tok3, dst0, dst1, dst2, dst3,
                   xbuf, sem):
    wid = lax.axis_index("s") * NC + lax.axis_index("c")
    iota16 = lax.iota(jnp.int32, 16)
    pltpu.sync_copy(eid_hbm, eid_v)

    def scan_body(k, cnt):
        ev = eid_v[pl.ds(k * 16, 16)]
        base = cnt.at[ev].get(mode="promise_in_bounds")
        rank = jnp.zeros((16,), jnp.int32)
        newcnt = cnt
        for e in range(E):
            m = ev == e
            inc = jnp.where(m, 1, 0).astype(jnp.int32)
            cs = jnp.cumsum(inc)
            rank = rank + jnp.where(m, cs, 0)
            tot = jnp.sum(inc)
            newcnt = newcnt + jnp.where(iota16 == e, tot, 0)
        pos_v[pl.ds(k * 16, 16)] = base + rank - 1
        return newcnt

    cnt = lax.fori_loop(0, NP // 16, scan_body,
                        jnp.zeros((16,), jnp.int32))
    aligned = ((cnt + (TB - 1)) // TB) * TB
    incl = jnp.cumsum(aligned)
    start = incl - aligned

    pbase = wid * PPT
    tok = (tok0, tok1, tok2, tok3)
    dst = (dst0, dst1, dst2, dst3)
    for c in range(4):
        for hh in range(2):
            off = pbase + c * 32 + hh * 16
            ev = eid_v[pl.ds(off, 16)]
            posv = pos_v[pl.ds(off, 16)]
            destv = start.at[ev].get(mode="promise_in_bounds") + posv
            dst[c][pl.ds(hh * 16, 16)] = destv
            pv = off + iota16
            tok[c][pl.ds(hh * 16, 16)] = jnp.bitwise_and(pv, S - 1)
        pltpu.async_copy(x_hbm.at[tok[c]], xbuf, sem).wait()
        pltpu.async_copy(xbuf, xg_hbm.at[dst[c]], sem).wait()
        pltpu.sync_copy(dst[c], row_hbm.at[pl.ds(pbase + c * 32, 32)])

    @pl.when(wid == 0)
    def _():
        nb_vec = aligned // TB
        bstart_vec = start // TB
        e_of_g = iota16 // 2
        sg_of_g = jnp.bitwise_and(iota16, 1)
        nb_g = nb_vec.at[e_of_g].get(mode="promise_in_bounds")
        bb_g = (bstart_vec.at[e_of_g].get(mode="promise_in_bounds")
                + sg_of_g * NBCAP)
        nbs_g = jnp.clip(nb_g - sg_of_g * NBCAP, 0, NBCAP)
        steps_g = NIF * nbs_g
        cum_g = jnp.cumsum(steps_g)
        segstart_g = cum_g - steps_g
        total = cum_g[15]
        laste = jnp.int32(0)
        lastb = jnp.int32(0)
        lastblk = jnp.int32(0)
        for g in range(16):
            ne = nbs_g[g] > 0
            laste = jnp.where(ne, jnp.int32(g // 2), laste)
            lastb = jnp.where(ne, nbs_g[g] - 1, lastb)
            lastblk = jnp.where(ne, bb_g[g] + nbs_g[g] - 1, lastblk)
        def colbody(ci, carry):
            sv = ci * 16 + iota16
            e_col = jnp.zeros((16,), jnp.int32)
            i_col = jnp.zeros((16,), jnp.int32)
            b_col = jnp.zeros((16,), jnp.int32)
            blk_col = jnp.zeros((16,), jnp.int32)
            for g in range(16):
                ss = segstart_g[g]
                nbs_s = jnp.maximum(nbs_g[g], 1)
                m = (sv >= ss) & (sv < ss + NIF * nbs_g[g])
                k = sv - ss
                i_val = k // nbs_s
                b_val = k - i_val * nbs_s
                e_col = jnp.where(m, jnp.int32(g // 2), e_col)
                i_col = jnp.where(m, i_val, i_col)
                b_col = jnp.where(m, b_val, b_col)
                blk_col = jnp.where(m, bb_g[g] + b_val, blk_col)
            valid = sv < total
            sl = pl.ds(ci * 16, 16)
            schedv[0, sl] = jnp.where(valid, e_col, laste)
            schedv[1, sl] = jnp.where(valid, i_col, NIF - 1)
            schedv[2, sl] = jnp.where(valid, b_col, lastb)
            schedv[3, sl] = jnp.where(valid, blk_col, lastblk)
            schedv[4, sl] = jnp.where(valid, 1, 0)
            return carry

        lax.fori_loop(0, NSTEPS // 16, colbody, 0)
        pltpu.sync_copy(schedv, sched_hbm)


def _dispatch(x, eid_flat):
    mesh = plsc.VectorSubcoreMesh(core_axis_name="c", subcore_axis_name="s")
    f = pl.kernel(
        _dispatch_body,
        out_type=[
            jax.ShapeDtypeStruct((CAP, H), jnp.float32),
            jax.ShapeDtypeStruct((NP,), jnp.int32),
            jax.ShapeDtypeStruct((5, NSTEPS), jnp.int32),
        ],
        mesh=mesh,
        scratch_types=[
            pltpu.VMEM((NP,), jnp.int32),
            pltpu.VMEM((NP,), jnp.int32),
            pltpu.VMEM((5, NSTEPS), jnp.int32),
            pltpu.VMEM((32,), jnp.int32),
            pltpu.VMEM((32,), jnp.int32),
            pltpu.VMEM((32,), jnp.int32),
            pltpu.VMEM((32,), jnp.int32),
            pltpu.VMEM((32,), jnp.int32),
            pltpu.VMEM((32,), jnp.int32),
            pltpu.VMEM((32,), jnp.int32),
            pltpu.VMEM((32,), jnp.int32),
            pltpu.VMEM((32, H), jnp.float32),
            pltpu.SemaphoreType.DMA,
        ],
        compiler_params=pltpu.CompilerParams(needs_layout_passes=False),
    )
    return f(x, eid_flat)


def _ffn_chunk(x, gw, uw, dw):
    g = jax.lax.dot_general(x, gw, (((1,), (1,)), ((), ())),
                            preferred_element_type=jnp.float32)
    g = g * jax.nn.sigmoid(g)
    u = jax.lax.dot_general(x, uw, (((1,), (1,)), ((), ())),
                            preferred_element_type=jnp.float32)
    h = g * u
    return jax.lax.dot_general(h, dw, (((1,), (1,)), ((), ())),
                               preferred_element_type=jnp.float32)


def _ffn_chunk_bf16(x, gw, uw, dw):
    xb = x.astype(jnp.bfloat16)
    g = jax.lax.dot_general(xb, gw.astype(jnp.bfloat16),
                            (((1,), (1,)), ((), ())),
                            preferred_element_type=jnp.float32)
    g = g * jax.nn.sigmoid(g)
    u = jax.lax.dot_general(xb, uw.astype(jnp.bfloat16),
                            (((1,), (1,)), ((), ())),
                            preferred_element_type=jnp.float32)
    h = g * u
    return jax.lax.dot_general(h.astype(jnp.bfloat16),
                               dw.astype(jnp.bfloat16),
                               (((1,), (1,)), ((), ())),
                               preferred_element_type=jnp.float32)


def _gffn_body(sched_ref, xg_ref, gw_ref, uw_ref, dw_ref, yg_ref,
               acc, xsc, sem, sem2):
    s = pl.program_id(0)
    i = sched_ref[1, s]
    ba = sched_ref[2, s]
    blk = sched_ref[3, s]
    valid = sched_ref[4, s]

    @pl.when(valid == 1)
    def _():
        row = pl.multiple_of(ba * TB, TB)
        grow = pl.multiple_of(blk * TB, TB)

        @pl.when(i == 0)
        def _():
            cp = pltpu.make_async_copy(
                xg_ref.at[pl.ds(grow, TB), :],
                xsc.at[pl.ds(row, TB), :], sem2)
            cp.start()
            cp.wait()

        y = _ffn_chunk(xsc[pl.ds(row, TB), :], gw_ref[0], uw_ref[0],
                       dw_ref[0])

        @pl.when(i == 0)
        def _():
            acc[pl.ds(row, TB), :] = y

        @pl.when(i > 0)
        def _():
            acc[pl.ds(row, TB), :] = acc[pl.ds(row, TB), :] + y

        @pl.when(i == NIF - 1)
        def _():
            cp = pltpu.make_async_copy(
                acc.at[pl.ds(row, TB), :],
                yg_ref.at[pl.ds(grow, TB), :], sem)
            cp.start()
            cp.wait()


def _gffn(sched, xg, gate_w, up_w, down_w):
    grid_spec = pltpu.PrefetchScalarGridSpec(
        num_scalar_prefetch=1,
        grid=(NSTEPS,),
        in_specs=[
            pl.BlockSpec(memory_space=pl.ANY),
            pl.BlockSpec((1, ICF, H), lambda s, sc: (sc[0, s], sc[1, s], 0)),
            pl.BlockSpec((1, ICF, H), lambda s, sc: (sc[0, s], sc[1, s], 0)),
            pl.BlockSpec((1, H, ICF), lambda s, sc: (sc[0, s], 0, sc[1, s])),
        ],
        out_specs=pl.BlockSpec(memory_space=pl.ANY),
        scratch_shapes=[
            pltpu.VMEM((NBCAP * TB, H), jnp.float32),
            pltpu.VMEM((NBCAP * TB, H), jnp.float32),
            pltpu.SemaphoreType.DMA,
            pltpu.SemaphoreType.DMA,
        ],
    )
    return pl.pallas_call(
        _gffn_body,
        grid_spec=grid_spec,
        out_shape=jax.ShapeDtypeStruct((CAP, H), jnp.float32),
        compiler_params=pltpu.CompilerParams(
            vmem_limit_bytes=60 * 1024 * 1024),
    )(sched, xg, gate_w, up_w, down_w)


def _shared_body(sx_ref, gw_ref, uw_ref, dw_ref, sg_ref, out_ref):
    i = pl.program_id(0)
    for ts in range(S // TSUB):
        sl = pl.ds(ts * TSUB, TSUB)
        y = _ffn_chunk_bf16(sx_ref[sl, :], gw_ref[...], uw_ref[...],
                            dw_ref[...])

        @pl.when(i == 0)
        def _():
            out_ref[sl, :] = y

        @pl.when(i > 0)
        def _():
            out_ref[sl, :] = out_ref[sl, :] + y

        @pl.when(i == NIS - 1)
        def _():
            sig = jax.nn.sigmoid(sg_ref[0, 0])
            out_ref[sl, :] = out_ref[sl, :] * sig


def _shared(sx, s_gate_w, s_up_w, s_down_w, shared_gate):
    return pl.pallas_call(
        _shared_body,
        grid=(NIS,),
        in_specs=[
            pl.BlockSpec((S, H), lambda i: (0, 0)),
            pl.BlockSpec((ICS, H), lambda i: (i, 0)),
            pl.BlockSpec((ICS, H), lambda i: (i, 0)),
            pl.BlockSpec((H, ICS), lambda i: (0, i)),
            pl.BlockSpec((1, 1), lambda i: (0, 0)),
        ],
        out_specs=pl.BlockSpec((S, H), lambda i: (0, 0)),
        out_shape=jax.ShapeDtypeStruct((S, H), jnp.float32),
        compiler_params=pltpu.CompilerParams(
            vmem_limit_bytes=62 * 1024 * 1024),
    )(sx, s_gate_w, s_up_w, s_down_w, shared_gate.reshape(1, 1))


CHT = 8
NCH = TPT // CHT


def _combine_body(yg_hbm, row_hbm, pw_hbm, sout_hbm, out_hbm,
                  idx0, idx1, w0v, w1v,
                  r0a, r1a, oba, r0b, r1b, obb, sema, semb, semo):
    wid = lax.axis_index("s") * NC + lax.axis_index("c")
    base = wid * TPT
    pltpu.sync_copy(row_hbm.at[pl.ds(base, TPT)], idx0)
    pltpu.sync_copy(row_hbm.at[pl.ds(S + base, TPT)], idx1)
    pltpu.sync_copy(pw_hbm.at[pl.ds(base, TPT)], w0v)
    pltpu.sync_copy(pw_hbm.at[pl.ds(S + base, TPT)], w1v)
    bufs = ((r0a, r1a, oba, sema), (r0b, r1b, obb, semb))

    def issue(c):
        r0, r1, ob, sem = bufs[c % 2]
        tb = base + c * CHT
        pltpu.async_copy(yg_hbm.at[idx0.at[pl.ds(c * CHT, CHT)]], r0, sem)
        pltpu.async_copy(yg_hbm.at[idx1.at[pl.ds(c * CHT, CHT)]], r1, sem)
        pltpu.async_copy(sout_hbm.at[pl.ds(tb, CHT)], ob, sem)

    issue(0)
    for c in range(NCH):
        r0, r1, ob, sem = bufs[c % 2]
        if c + 1 < NCH:
            issue(c + 1)
        pltpu.make_async_copy(yg_hbm.at[idx0.at[pl.ds(c * CHT, CHT)]],
                              r0, sem).wait()
        pltpu.make_async_copy(yg_hbm.at[idx1.at[pl.ds(c * CHT, CHT)]],
                              r1, sem).wait()
        pltpu.make_async_copy(sout_hbm.at[pl.ds(base + c * CHT, CHT)],
                              ob, sem).wait()
        for j in range(CHT):
            wsl = pl.ds((c * CHT + j) // 16 * 16, 16)
            w0s = w0v[wsl][(c * CHT + j) % 16]
            w1s = w1v[wsl][(c * CHT + j) % 16]

            def cb(ci, _, j=j, w0s=w0s, w1s=w1s, r0=r0, r1=r1, ob=ob):
                sl = pl.ds(ci * 16, 16)
                ob[j, sl] = (ob[j, sl] + w0s * r0[j, sl] + w1s * r1[j, sl])
                return 0

            lax.fori_loop(0, H // 16, cb, 0)
        pltpu.async_copy(ob, out_hbm.at[pl.ds(base + c * CHT, CHT)],
                         semo).wait()


def _combine(yg, pair_row, pw_flat, sout):
    mesh = plsc.VectorSubcoreMesh(core_axis_name="c", subcore_axis_name="s")
    f = pl.kernel(
        _combine_body,
        out_type=jax.ShapeDtypeStruct((S, H), jnp.float32),
        mesh=mesh,
        scratch_types=[
            pltpu.VMEM((TPT,), jnp.int32),
            pltpu.VMEM((TPT,), jnp.int32),
            pltpu.VMEM((TPT,), jnp.float32),
            pltpu.VMEM((TPT,), jnp.float32),
            pltpu.VMEM((CHT, H), jnp.float32),
            pltpu.VMEM((CHT, H), jnp.float32),
            pltpu.VMEM((CHT, H), jnp.float32),
            pltpu.VMEM((CHT, H), jnp.float32),
            pltpu.VMEM((CHT, H), jnp.float32),
            pltpu.VMEM((CHT, H), jnp.float32),
            pltpu.SemaphoreType.DMA,
            pltpu.SemaphoreType.DMA,
            pltpu.SemaphoreType.DMA,
        ],
        compiler_params=pltpu.CompilerParams(needs_layout_passes=False),
    )
    return f(yg, pair_row, pw_flat, sout)


def kernel(hidden_states, ln_g, ln_b, router_w, expert_bias, gate_w, up_w,
           down_w, s_ln_g, s_ln_b, s_gate_w, s_up_w, s_down_w, shared_gate):
    B, S_, H_ = hidden_states.shape
    x = hidden_states.reshape(-1, H_)
    pair_eid, pair_w, zpart, sx = _router(x, ln_g, ln_b, s_ln_g, s_ln_b,
                                          router_w, expert_bias)
    xg, pair_row, sched = _dispatch(x, pair_eid.reshape(NP))
    sout = _shared(sx, s_gate_w, s_up_w, s_down_w, shared_gate)
    yg = _gffn(sched, xg, gate_w, up_w, down_w)
    final = _combine(yg, pair_row, pair_w.reshape(NP), sout)
    z_loss = jnp.sum(zpart) / S_ * 0.0001
    return (final.reshape(B, S_, H_), z_loss)


# speedup vs baseline: 1.0671x; 1.0102x over previous
import functools

import jax
import jax.numpy as jnp
from jax import lax
from jax.experimental import pallas as pl
from jax.experimental.pallas import tpu as pltpu
from jax.experimental.pallas import tpu_sc as plsc

EPS = 1e-05
S = 2048
H = 2048
I = 4096
E = 8
K = 2
NP = K * S

TBR = 256
GR = S // TBR

TB = 256
NBLK = 23
CAP = NBLK * TB
NBCAP = 8
ICF = 512
NIF = I // ICF
NSTEPS = 192

TSUB = 512
ICS = 512
NIS = I // ICS

NC = 2
NS = 16
NW = NC * NS
PPT = NP // NW
TPT = S // NW


def _router_body(x_ref, g_ref, b_ref, sg_ref, sb_ref, rw_ref, bias_ref,
                 eid_ref, pw_ref, z_ref, sx_ref):
    b = pl.program_id(0)
    x = x_ref[...]
    m = jnp.mean(x, axis=1, keepdims=True)
    v = jnp.mean((x - m) ** 2, axis=1, keepdims=True)
    xc = (x - m) / jnp.sqrt(v + 1e-05)
    xn = xc * g_ref[...] + b_ref[...]
    sx_ref[...] = xc * sg_ref[...] + sb_ref[...]
    logits = jax.lax.dot_general(xn, rw_ref[...],
                                 (((1,), (1,)), ((), ())),
                                 preferred_element_type=jnp.float32)
    logits = logits + bias_ref[...]
    lmax = jnp.max(logits, axis=1, keepdims=True)
    ex = jnp.exp(logits - lmax)
    sex = jnp.sum(ex, axis=1, keepdims=True)
    lse = lmax + jnp.log(sex)
    p = ex / sex

    iota = lax.broadcasted_iota(jnp.int32, (TBR, E), 1)
    m1 = jnp.max(p, axis=1, keepdims=True)
    i1 = jnp.min(jnp.where(p == m1, iota, E), axis=1, keepdims=True)
    p2 = jnp.where(iota == i1, -jnp.inf, p)
    m2 = jnp.max(p2, axis=1, keepdims=True)
    i2 = jnp.min(jnp.where(p2 == m2, iota, E), axis=1, keepdims=True)
    ssum = jnp.clip(m1 + m2, EPS, None)
    w1 = m1 / ssum
    w2 = m2 / ssum
    eid_ref[0, pl.ds(b * TBR, TBR)] = i1[:, 0]
    eid_ref[1, pl.ds(b * TBR, TBR)] = i2[:, 0]
    pw_ref[0, pl.ds(b * TBR, TBR)] = w1[:, 0]
    pw_ref[1, pl.ds(b * TBR, TBR)] = w2[:, 0]
    zrow = jnp.sum(lse * lse)
    lane = lax.broadcasted_iota(jnp.int32, (1, 128), 1)
    z_ref[pl.ds(b, 1), :] = jnp.where(lane == 0, zrow, 0.0)


def _router(x, ln_g, ln_b, s_ln_g, s_ln_b, router_w, expert_bias):
    return pl.pallas_call(
        _router_body,
        grid=(GR,),
        in_specs=[
            pl.BlockSpec((TBR, H), lambda b: (b, 0)),
            pl.BlockSpec((1, H), lambda b: (0, 0)),
            pl.BlockSpec((1, H), lambda b: (0, 0)),
            pl.BlockSpec((1, H), lambda b: (0, 0)),
            pl.BlockSpec((1, H), lambda b: (0, 0)),
            pl.BlockSpec((E, H), lambda b: (0, 0)),
            pl.BlockSpec((1, E), lambda b: (0, 0)),
        ],
        out_specs=[
            pl.BlockSpec((K, S), lambda b: (0, 0)),
            pl.BlockSpec((K, S), lambda b: (0, 0)),
            pl.BlockSpec((GR, 128), lambda b: (0, 0)),
            pl.BlockSpec((TBR, H), lambda b: (b, 0)),
        ],
        out_shape=[
            jax.ShapeDtypeStruct((K, S), jnp.int32),
            jax.ShapeDtypeStruct((K, S), jnp.float32),
            jax.ShapeDtypeStruct((GR, 128), jnp.float32),
            jax.ShapeDtypeStruct((S, H), jnp.float32),
        ],
    )(x, ln_g.reshape(1, H), ln_b.reshape(1, H), s_ln_g.reshape(1, H),
      s_ln_b.reshape(1, H), router_w, expert_bias.reshape(1, E))


def _dispatch_body(x_hbm, eid_hbm, xg_hbm, row_hbm, sched_hbm,
                   eid_v, pos_v, schedv,
                   tok0, tok1, tok2, tok3, dst0, dst1, dst2, dst3,
                   xbuf, sem):
    wid = lax.axis_index("s") * NC + lax.axis_index("c")
    iota16 = lax.iota(jnp.int32, 16)
    pltpu.sync_copy(eid_hbm, eid_v)

    def scan_body(k, cnt):
        ev = eid_v[pl.ds(k * 16, 16)]
        base = cnt.at[ev].get(mode="promise_in_bounds")
        rank = jnp.zeros((16,), jnp.int32)
        newcnt = cnt
        for e in range(E):
            m = ev == e
            inc = jnp.where(m, 1, 0).astype(jnp.int32)
            cs = jnp.cumsum(inc)
            rank = rank + jnp.where(m, cs, 0)
            tot = jnp.sum(inc)
            newcnt = newcnt + jnp.where(iota16 == e, tot, 0)
        pos_v[pl.ds(k * 16, 16)] = base + rank - 1
        return newcnt

    cnt = lax.fori_loop(0, NP // 16, scan_body,
                        jnp.zeros((16,), jnp.int32))
    aligned = ((cnt + (TB - 1)) // TB) * TB
    incl = jnp.cumsum(aligned)
    start = incl - aligned

    pbase = wid * PPT
    tok = (tok0, tok1, tok2, tok3)
    dst = (dst0, dst1, dst2, dst3)
    for c in range(4):
        for hh in range(2):
            off = pbase + c * 32 + hh * 16
            ev = eid_v[pl.ds(off, 16)]
            posv = pos_v[pl.ds(off, 16)]
            destv = start.at[ev].get(mode="promise_in_bounds") + posv
            dst[c][pl.ds(hh * 16, 16)] = destv
            pv = off + iota16
            tok[c][pl.ds(hh * 16, 16)] = jnp.bitwise_and(pv, S - 1)
        pltpu.async_copy(x_hbm.at[tok[c]], xbuf, sem).wait()
        pltpu.async_copy(xbuf, xg_hbm.at[dst[c]], sem).wait()
        pltpu.sync_copy(dst[c], row_hbm.at[pl.ds(pbase + c * 32, 32)])

    @pl.when(wid == 0)
    def _():
        nb_vec = aligned // TB
        bstart_vec = start // TB
        e_of_g = iota16 // 2
        sg_of_g = jnp.bitwise_and(iota16, 1)
        nb_g = nb_vec.at[e_of_g].get(mode="promise_in_bounds")
        bb_g = (bstart_vec.at[e_of_g].get(mode="promise_in_bounds")
                + sg_of_g * NBCAP)
        nbs_g = jnp.clip(nb_g - sg_of_g * NBCAP, 0, NBCAP)
        steps_g = NIF * nbs_g
        cum_g = jnp.cumsum(steps_g)
        segstart_g = cum_g - steps_g
        total = cum_g[15]
        laste = jnp.int32(0)
        lastb = jnp.int32(0)
        lastblk = jnp.int32(0)
        for g in range(16):
            ne = nbs_g[g] > 0
            laste = jnp.where(ne, jnp.int32(g // 2), laste)
            lastb = jnp.where(ne, nbs_g[g] - 1, lastb)
            lastblk = jnp.where(ne, bb_g[g] + nbs_g[g] - 1, lastblk)
        def colbody(ci, carry):
            sv = ci * 16 + iota16
            e_col = jnp.zeros((16,), jnp.int32)
            i_col = jnp.zeros((16,), jnp.int32)
            b_col = jnp.zeros((16,), jnp.int32)
            blk_col = jnp.zeros((16,), jnp.int32)
            for g in range(16):
                ss = segstart_g[g]
                nbs_s = jnp.maximum(nbs_g[g], 1)
                m = (sv >= ss) & (sv < ss + NIF * nbs_g[g])
                k = sv - ss
                i_val = k // nbs_s
                b_val = k - i_val * nbs_s
                e_col = jnp.where(m, jnp.int32(g // 2), e_col)
                i_col = jnp.where(m, i_val, i_col)
                b_col = jnp.where(m, b_val, b_col)
                blk_col = jnp.where(m, bb_g[g] + b_val, blk_col)
            valid = sv < total
            sl = pl.ds(ci * 16, 16)
            schedv[0, sl] = jnp.where(valid, e_col, laste)
            schedv[1, sl] = jnp.where(valid, i_col, NIF - 1)
            schedv[2, sl] = jnp.where(valid, b_col, lastb)
            schedv[3, sl] = jnp.where(valid, blk_col, lastblk)
            schedv[4, sl] = jnp.where(valid, 1, 0)
            return carry

        lax.fori_loop(0, NSTEPS // 16, colbody, 0)
        pltpu.sync_copy(schedv, sched_hbm)


def _dispatch(x, eid_flat):
    mesh = plsc.VectorSubcoreMesh(core_axis_name="c", subcore_axis_name="s")
    f = pl.kernel(
        _dispatch_body,
        out_type=[
            jax.ShapeDtypeStruct((CAP, H), jnp.float32),
            jax.ShapeDtypeStruct((NP,), jnp.int32),
            jax.ShapeDtypeStruct((5, NSTEPS), jnp.int32),
        ],
        mesh=mesh,
        scratch_types=[
            pltpu.VMEM((NP,), jnp.int32),
            pltpu.VMEM((NP,), jnp.int32),
            pltpu.VMEM((5, NSTEPS), jnp.int32),
            pltpu.VMEM((32,), jnp.int32),
            pltpu.VMEM((32,), jnp.int32),
            pltpu.VMEM((32,), jnp.int32),
            pltpu.VMEM((32,), jnp.int32),
            pltpu.VMEM((32,), jnp.int32),
            pltpu.VMEM((32,), jnp.int32),
            pltpu.VMEM((32,), jnp.int32),
            pltpu.VMEM((32,), jnp.int32),
            pltpu.VMEM((32, H), jnp.float32),
            pltpu.SemaphoreType.DMA,
        ],
        compiler_params=pltpu.CompilerParams(needs_layout_passes=False),
    )
    return f(x, eid_flat)


def _ffn_chunk(x, gw, uw, dw):
    g = jax.lax.dot_general(x, gw, (((1,), (1,)), ((), ())),
                            preferred_element_type=jnp.float32)
    g = g * jax.nn.sigmoid(g)
    u = jax.lax.dot_general(x, uw, (((1,), (1,)), ((), ())),
                            preferred_element_type=jnp.float32)
    h = g * u
    return jax.lax.dot_general(h, dw, (((1,), (1,)), ((), ())),
                               preferred_element_type=jnp.float32)


def _ffn_chunk_bf16(x, gw, uw, dw):
    xb = x.astype(jnp.bfloat16)
    g = jax.lax.dot_general(xb, gw.astype(jnp.bfloat16),
                            (((1,), (1,)), ((), ())),
                            preferred_element_type=jnp.float32)
    g = g * jax.nn.sigmoid(g)
    u = jax.lax.dot_general(xb, uw.astype(jnp.bfloat16),
                            (((1,), (1,)), ((), ())),
                            preferred_element_type=jnp.float32)
    h = g * u
    return jax.lax.dot_general(h.astype(jnp.bfloat16),
                               dw.astype(jnp.bfloat16),
                               (((1,), (1,)), ((), ())),
                               preferred_element_type=jnp.float32)


def _gffn_body(sched_ref, xg_ref, gw_ref, uw_ref, dw_ref, yg_ref,
               acc, xsc, sem, sem2):
    s = pl.program_id(0)
    i = sched_ref[1, s]
    ba = sched_ref[2, s]
    blk = sched_ref[3, s]
    valid = sched_ref[4, s]

    @pl.when(valid == 1)
    def _():
        row = pl.multiple_of(ba * TB, TB)
        grow = pl.multiple_of(blk * TB, TB)

        @pl.when(i == 0)
        def _():
            cp = pltpu.make_async_copy(
                xg_ref.at[pl.ds(grow, TB), :],
                xsc.at[pl.ds(row, TB), :], sem2)
            cp.start()
            cp.wait()

        y = _ffn_chunk(xsc[pl.ds(row, TB), :], gw_ref[0], uw_ref[0],
                       dw_ref[0])

        @pl.when(i == 0)
        def _():
            acc[pl.ds(row, TB), :] = y

        @pl.when(i > 0)
        def _():
            acc[pl.ds(row, TB), :] = acc[pl.ds(row, TB), :] + y

        @pl.when(i == NIF - 1)
        def _():
            cp = pltpu.make_async_copy(
                acc.at[pl.ds(row, TB), :],
                yg_ref.at[pl.ds(grow, TB), :], sem)
            cp.start()
            cp.wait()


def _gffn(sched, xg, gate_w, up_w, down_w):
    grid_spec = pltpu.PrefetchScalarGridSpec(
        num_scalar_prefetch=1,
        grid=(NSTEPS,),
        in_specs=[
            pl.BlockSpec(memory_space=pl.ANY),
            pl.BlockSpec((1, ICF, H), lambda s, sc: (sc[0, s], sc[1, s], 0)),
            pl.BlockSpec((1, ICF, H), lambda s, sc: (sc[0, s], sc[1, s], 0)),
            pl.BlockSpec((1, H, ICF), lambda s, sc: (sc[0, s], 0, sc[1, s])),
        ],
        out_specs=pl.BlockSpec(memory_space=pl.ANY),
        scratch_shapes=[
            pltpu.VMEM((NBCAP * TB, H), jnp.float32),
            pltpu.VMEM((NBCAP * TB, H), jnp.float32),
            pltpu.SemaphoreType.DMA,
            pltpu.SemaphoreType.DMA,
        ],
    )
    return pl.pallas_call(
        _gffn_body,
        grid_spec=grid_spec,
        out_shape=jax.ShapeDtypeStruct((CAP, H), jnp.float32),
        compiler_params=pltpu.CompilerParams(
            vmem_limit_bytes=60 * 1024 * 1024),
    )(sched, xg, gate_w, up_w, down_w)


def _shared_body(sx_ref, gw_ref, uw_ref, dw_ref, sg_ref, out_ref):
    i = pl.program_id(0)
    for ts in range(S // TSUB):
        sl = pl.ds(ts * TSUB, TSUB)
        y = _ffn_chunk_bf16(sx_ref[sl, :], gw_ref[...], uw_ref[...],
                            dw_ref[...])

        @pl.when(i == 0)
        def _():
            out_ref[sl, :] = y

        @pl.when(i > 0)
        def _():
            out_ref[sl, :] = out_ref[sl, :] + y

        @pl.when(i == NIS - 1)
        def _():
            sig = jax.nn.sigmoid(sg_ref[0, 0])
            out_ref[sl, :] = out_ref[sl, :] * sig


def _shared(sx, s_gate_w, s_up_w, s_down_w, shared_gate):
    return pl.pallas_call(
        _shared_body,
        grid=(NIS,),
        in_specs=[
            pl.BlockSpec((S, H), lambda i: (0, 0)),
            pl.BlockSpec((ICS, H), lambda i: (i, 0)),
            pl.BlockSpec((ICS, H), lambda i: (i, 0)),
            pl.BlockSpec((H, ICS), lambda i: (0, i)),
            pl.BlockSpec((1, 1), lambda i: (0, 0)),
        ],
        out_specs=pl.BlockSpec((S, H), lambda i: (0, 0)),
        out_shape=jax.ShapeDtypeStruct((S, H), jnp.float32),
        compiler_params=pltpu.CompilerParams(
            vmem_limit_bytes=62 * 1024 * 1024),
    )(sx, s_gate_w, s_up_w, s_down_w, shared_gate.reshape(1, 1))


CHT = 8
NCH = TPT // CHT


def _combine_body(yg_hbm, row_hbm, pw_hbm, sout_hbm, out_hbm,
                  idx0, idx1, w0v, w1v,
                  r0a, r1a, oba, r0b, r1b, obb, sema, semb, semo):
    wid = lax.axis_index("s") * NC + lax.axis_index("c")
    base = wid * TPT
    pltpu.sync_copy(row_hbm.at[pl.ds(base, TPT)], idx0)
    pltpu.sync_copy(row_hbm.at[pl.ds(S + base, TPT)], idx1)
    pltpu.sync_copy(pw_hbm.at[pl.ds(base, TPT)], w0v)
    pltpu.sync_copy(pw_hbm.at[pl.ds(S + base, TPT)], w1v)
    bufs = ((r0a, r1a, oba, sema), (r0b, r1b, obb, semb))

    def issue(c):
        r0, r1, ob, sem = bufs[c % 2]
        tb = base + c * CHT
        pltpu.async_copy(yg_hbm.at[idx0.at[pl.ds(c * CHT, CHT)]], r0, sem)
        pltpu.async_copy(yg_hbm.at[idx1.at[pl.ds(c * CHT, CHT)]], r1, sem)
        pltpu.async_copy(sout_hbm.at[pl.ds(tb, CHT)], ob, sem)

    issue(0)
    for c in range(NCH):
        r0, r1, ob, sem = bufs[c % 2]
        if c + 1 < NCH:
            issue(c + 1)
        pltpu.make_async_copy(yg_hbm.at[idx0.at[pl.ds(c * CHT, CHT)]],
                              r0, sem).wait()
        pltpu.make_async_copy(yg_hbm.at[idx1.at[pl.ds(c * CHT, CHT)]],
                              r1, sem).wait()
        pltpu.make_async_copy(sout_hbm.at[pl.ds(base + c * CHT, CHT)],
                              ob, sem).wait()
        for j in range(CHT):
            wsl = pl.ds((c * CHT + j) // 16 * 16, 16)
            w0s = w0v[wsl][(c * CHT + j) % 16]
            w1s = w1v[wsl][(c * CHT + j) % 16]

            def cb(ci, _, j=j, w0s=w0s, w1s=w1s, r0=r0, r1=r1, ob=ob):
                for u in range(4):
                    sl = pl.ds(ci * 64 + u * 16, 16)
                    ob[j, sl] = (ob[j, sl] + w0s * r0[j, sl]
                                 + w1s * r1[j, sl])
                return 0

            lax.fori_loop(0, H // 64, cb, 0)
        pltpu.async_copy(ob, out_hbm.at[pl.ds(base + c * CHT, CHT)],
                         semo).wait()


def _combine(yg, pair_row, pw_flat, sout):
    mesh = plsc.VectorSubcoreMesh(core_axis_name="c", subcore_axis_name="s")
    f = pl.kernel(
        _combine_body,
        out_type=jax.ShapeDtypeStruct((S, H), jnp.float32),
        mesh=mesh,
        scratch_types=[
            pltpu.VMEM((TPT,), jnp.int32),
            pltpu.VMEM((TPT,), jnp.int32),
            pltpu.VMEM((TPT,), jnp.float32),
            pltpu.VMEM((TPT,), jnp.float32),
            pltpu.VMEM((CHT, H), jnp.float32),
            pltpu.VMEM((CHT, H), jnp.float32),
            pltpu.VMEM((CHT, H), jnp.float32),
            pltpu.VMEM((CHT, H), jnp.float32),
            pltpu.VMEM((CHT, H), jnp.float32),
            pltpu.VMEM((CHT, H), jnp.float32),
            pltpu.SemaphoreType.DMA,
            pltpu.SemaphoreType.DMA,
            pltpu.SemaphoreType.DMA,
        ],
        compiler_params=pltpu.CompilerParams(needs_layout_passes=False),
    )
    return f(yg, pair_row, pw_flat, sout)


def kernel(hidden_states, ln_g, ln_b, router_w, expert_bias, gate_w, up_w,
           down_w, s_ln_g, s_ln_b, s_gate_w, s_up_w, s_down_w, shared_gate):
    B, S_, H_ = hidden_states.shape
    x = hidden_states.reshape(-1, H_)
    pair_eid, pair_w, zpart, sx = _router(x, ln_g, ln_b, s_ln_g, s_ln_b,
                                          router_w, expert_bias)
    xg, pair_row, sched = _dispatch(x, pair_eid.reshape(NP))
    sout = _shared(sx, s_gate_w, s_up_w, s_down_w, shared_gate)
    yg = _gffn(sched, xg, gate_w, up_w, down_w)
    final = _combine(yg, pair_row, pair_w.reshape(NP), sout)
    z_loss = jnp.sum(zpart) / S_ * 0.0001
    return (final.reshape(B, S_, H_), z_loss)
